# baseline (speedup 1.0000x reference)
import functools

import jax
import jax.numpy as jnp
from jax import lax
from jax.experimental import pallas as pl
from jax.experimental.pallas import tpu as pltpu
from jax.experimental.pallas import tpu_sc as plsc

N_NODES = 10000
N_EDGES = 320000
F = 128
FH = F // 2

NC = 2
NS = 16
L = 16

EDGES_PER_TILE = N_EDGES // NS
SUB = 80
NSUB = 2
CHUNK = NSUB * SUB
N_CHUNKS = EDGES_PER_TILE // CHUNK
NSLOT = 3
ROWS_PER_TILE = 624
TAIL_ROWS = N_NODES - NS * ROWS_PER_TILE


def _mm_body(x_ref, w_ref, o_ref):
    o_ref[0] = jnp.dot(x_ref[...], w_ref[0], preferred_element_type=jnp.float32)


def _matmul(x, Wh):
    return pl.pallas_call(
        _mm_body,
        grid=(2, 10),
        in_specs=[
            pl.BlockSpec((1000, F), lambda h, i: (i, 0)),
            pl.BlockSpec((1, F, FH), lambda h, i: (h, 0, 0)),
        ],
        out_specs=pl.BlockSpec((1, 1000, FH), lambda h, i: (h, i, 0)),
        out_shape=jax.ShapeDtypeStruct((NC, N_NODES, FH), jnp.float32),
    )(x, Wh)


_mesh = plsc.VectorSubcoreMesh(core_axis_name="c", subcore_axis_name="s")


@functools.partial(
    pl.kernel,
    out_type=jax.ShapeDtypeStruct((NC, N_NODES, FH), jnp.float32),
    mesh=_mesh,
    compiler_params=pltpu.CompilerParams(use_tc_tiling_on_sc=False),
    scratch_types=[
        pltpu.VMEM((NSLOT, NSUB, SUB), jnp.int32),
        pltpu.VMEM((NSLOT, NSUB, SUB), jnp.int32),
        pltpu.VMEM((NSLOT, NSUB, SUB), jnp.float32),
        pltpu.VMEM((NSLOT, CHUNK, FH), jnp.float32),
        pltpu.VMEM_SHARED((N_NODES, FH), jnp.float32),
        pltpu.SemaphoreType.DMA,
        pltpu.SemaphoreType.DMA,
        pltpu.SemaphoreType.DMA,
        pltpu.SemaphoreType.DMA,
    ],
)
def _spmm(sup, col, row, w, out, col3, row3, w3, gb3, acc, gsem0, gsem1, gsem2, isem):
    gsems = (gsem0, gsem1, gsem2)
    c = lax.axis_index("c")
    s = lax.axis_index("s")
    tile_base = s * EDGES_PER_TILE
    suph = sup.at[c]

    def idx_copies(i, slot):
        ds = []
        for sub in range(NSUB):
            base = tile_base + i * CHUNK + sub * SUB
            ds += [
                pltpu.make_async_copy(col.at[pl.ds(base, SUB)], col3.at[slot, sub], isem),
                pltpu.make_async_copy(row.at[pl.ds(base, SUB)], row3.at[slot, sub], isem),
                pltpu.make_async_copy(w.at[pl.ds(base, SUB)], w3.at[slot, sub], isem),
            ]
        return ds

    def idx_load(i, slot):
        for d in idx_copies(i, slot):
            d.start()

    def idx_wait(i, slot):
        for d in idx_copies(i, slot):
            d.wait()

    def gather_copies(slot):
        return [
            pltpu.make_async_copy(suph.at[col3.at[slot, sub]],
                                  gb3.at[slot, pl.ds(sub * SUB, SUB)], gsems[slot])
            for sub in range(NSUB)
        ]

    def gather_start(slot):
        for d in gather_copies(slot):
            d.start()

    def gather_wait(slot):
        for d in gather_copies(slot):
            d.wait()

    def scale(slot):
        for sub in range(NSUB):
            def body(j16, c2, sub=sub):
                wv = w3[slot, sub, pl.ds(j16 * L, L)]
                for k in range(L):
                    wj = jnp.broadcast_to(wv[k], (L,))
                    e = sub * SUB + j16 * L + k
                    for f in range(FH // L):
                        sl = pl.ds(f * L, L)
                        gb3[slot, e, sl] = gb3[slot, e, sl] * wj
                return c2

            lax.fori_loop(0, SUB // L, body, 0)

    def scatter_add(slot):
        for sub in range(NSUB):
            pltpu.sync_copy(gb3.at[slot, pl.ds(sub * SUB, SUB)],
                            acc.at[row3.at[slot, sub]], add=True)

    def step(i, slot, start2, load3):
        s2 = (slot + 2) % NSLOT
        if start2:
            idx_wait(i + 2, s2)
        gather_wait(slot)
        if start2:
            gather_start(s2)
        scale(slot)
        scatter_add(slot)
        if load3:
            idx_load(i + 3, slot)

    zero = jnp.zeros((L,), jnp.float32)

    def zrow(i, carry):
        for f in range(FH // L):
            gb3[0, i, pl.ds(f * L, L)] = zero
        return carry

    lax.fori_loop(0, CHUNK, zrow, 0)
    r0 = s * ROWS_PER_TILE
    for off in range(0, ROWS_PER_TILE, CHUNK):
        sz = min(CHUNK, ROWS_PER_TILE - off)
        pltpu.sync_copy(gb3.at[0, pl.ds(0, sz)], acc.at[pl.ds(r0 + off, sz)])

    @pl.when(s == NS - 1)
    def _zero_tail():
        pltpu.sync_copy(gb3.at[0, pl.ds(0, TAIL_ROWS)],
                        acc.at[pl.ds(NS * ROWS_PER_TILE, TAIL_ROWS)])

    plsc.subcore_barrier()

    idx_load(0, 0)
    idx_wait(0, 0)
    gather_start(0)
    idx_load(1, 1)
    idx_wait(1, 1)
    gather_start(1)
    idx_load(2, 2)

    n_triples = (N_CHUNKS - 5) // 3

    def triple(p, carry):
        i0 = 3 * p
        step(i0, 0, True, True)
        step(i0 + 1, 1, True, True)
        step(i0 + 2, 2, True, True)
        return carry

    lax.fori_loop(0, n_triples, triple, 0)

    i = 3 * n_triples
    for j in range(i, N_CHUNKS):
        step(j, j % NSLOT, j + 2 < N_CHUNKS, j + 3 < N_CHUNKS)

    plsc.subcore_barrier()

    for off in range(0, ROWS_PER_TILE, CHUNK):
        sz = min(CHUNK, ROWS_PER_TILE - off)
        pltpu.sync_copy(acc.at[pl.ds(r0 + off, sz)], gb3.at[0, pl.ds(0, sz)])
        pltpu.sync_copy(gb3.at[0, pl.ds(0, sz)], out.at[c, pl.ds(r0 + off, sz)])

    @pl.when(s == NS - 1)
    def _write_tail():
        t0 = NS * ROWS_PER_TILE
        pltpu.sync_copy(acc.at[pl.ds(t0, TAIL_ROWS)], gb3.at[0, pl.ds(0, TAIL_ROWS)])
        pltpu.sync_copy(gb3.at[0, pl.ds(0, TAIL_ROWS)], out.at[c, pl.ds(t0, TAIL_ROWS)])


def _comb_body(p_ref, b_ref, o_ref):
    o_ref[:, 0:FH] = p_ref[0] + b_ref[0]
    o_ref[:, FH:F] = p_ref[1] + b_ref[1]


def _combine(partials, b2):
    return pl.pallas_call(
        _comb_body,
        grid=(10,),
        in_specs=[
            pl.BlockSpec((NC, 1000, FH), lambda i: (0, i, 0)),
            pl.BlockSpec((NC, FH), lambda i: (0, 0)),
        ],
        out_specs=pl.BlockSpec((1000, F), lambda i: (i, 0)),
        out_shape=jax.ShapeDtypeStruct((N_NODES, F), jnp.float32),
    )(partials, b2)


def kernel(input, edge_index, edge_weight, W, b):
    ei = edge_index.astype(jnp.int32)
    row = ei[0]
    col = ei[1]
    Wh = jnp.stack([W[:, :FH], W[:, FH:]])
    sup = _matmul(input, Wh)
    partials = _spmm(sup, col, row, edge_weight)
    return _combine(partials, b.reshape(NC, FH))


# speedup vs baseline: 1.9646x; 1.9646x over previous
import functools

import jax
import jax.numpy as jnp
from jax import lax
from jax.experimental import pallas as pl
from jax.experimental.pallas import tpu as pltpu
from jax.experimental.pallas import tpu_sc as plsc

N_NODES = 10000
N_EDGES = 320000
F = 128

NC = 2
NS = 16
L = 16

EDGES_PER_TILE = N_EDGES // (NC * NS)
CHUNK = 80
N_CHUNKS = EDGES_PER_TILE // CHUNK
NSLOT = 4
ROWS_PER_TILE = 624
TAIL_ROWS = N_NODES - NS * ROWS_PER_TILE


def _mm_body(x_ref, w_ref, o_ref):
    o_ref[...] = jnp.dot(x_ref[...], w_ref[...], preferred_element_type=jnp.float32)


def _matmul(x, W):
    return pl.pallas_call(
        _mm_body,
        grid=(10,),
        in_specs=[
            pl.BlockSpec((1000, F), lambda i: (i, 0)),
            pl.BlockSpec((F, F), lambda i: (0, 0)),
        ],
        out_specs=pl.BlockSpec((1000, F), lambda i: (i, 0)),
        out_shape=jax.ShapeDtypeStruct((N_NODES, F), jnp.float32),
    )(x, W)


_mesh = plsc.VectorSubcoreMesh(core_axis_name="c", subcore_axis_name="s")


@functools.partial(
    pl.kernel,
    out_type=jax.ShapeDtypeStruct((NC, N_NODES, F), jnp.float32),
    mesh=_mesh,
    scratch_types=[
        pltpu.VMEM((NSLOT, 1, CHUNK), jnp.int32),
        pltpu.VMEM((NSLOT, 1, CHUNK), jnp.int32),
        pltpu.VMEM((NSLOT, 1, CHUNK), jnp.float32),
        pltpu.VMEM((NSLOT, CHUNK, F), jnp.float32),
        pltpu.VMEM_SHARED((N_NODES, F), jnp.float32),
        pltpu.SemaphoreType.DMA,
        pltpu.SemaphoreType.DMA,
        pltpu.SemaphoreType.DMA,
        pltpu.SemaphoreType.DMA,
        pltpu.SemaphoreType.DMA,
    ],
)
def _spmm(sup, col, row, w, out, col4, row4, w4, gb4, acc,
          gsem0, gsem1, gsem2, gsem3, isem):
    gsems = (gsem0, gsem1, gsem2, gsem3)
    c = lax.axis_index("c")
    s = lax.axis_index("s")
    gid = c * NS + s
    tile_base = gid * EDGES_PER_TILE

    def idx_copies(i, slot):
        base = tile_base + i * CHUNK
        return (
            pltpu.make_async_copy(col.at[pl.ds(base, CHUNK)], col4.at[slot, 0], isem),
            pltpu.make_async_copy(row.at[pl.ds(base, CHUNK)], row4.at[slot, 0], isem),
            pltpu.make_async_copy(w.at[pl.ds(base, CHUNK)], w4.at[slot, 0], isem),
        )

    def idx_load(i, slot):
        for d in idx_copies(i, slot):
            d.start()

    def idx_wait(i, slot):
        for d in idx_copies(i, slot):
            d.wait()

    def gather_copy(slot):
        return pltpu.make_async_copy(sup.at[col4.at[slot, 0]], gb4.at[slot],
                                     gsems[slot])

    def scale(slot):
        def body(j16, c2):
            wv = w4[slot, 0, pl.ds(j16 * L, L)]
            for k in range(L):
                wj = jnp.broadcast_to(wv[k], (L,))
                e = j16 * L + k
                for f in range(F // L):
                    sl = pl.ds(f * L, L)
                    gb4[slot, e, sl] = gb4[slot, e, sl] * wj
            return c2

        lax.fori_loop(0, CHUNK // L, body, 0)

    def scatter_add(slot):
        pltpu.sync_copy(gb4.at[slot], acc.at[row4.at[slot, 0]], add=True)

    def step(i, slot, start3, load4):
        s3 = (slot + 3) % NSLOT
        if start3:
            idx_wait(i + 3, s3)
        gather_copy(slot).wait()
        if start3:
            gather_copy(s3).start()
        scale(slot)
        scatter_add(slot)
        if load4:
            idx_load(i + 4, slot)

    zero = jnp.zeros((L,), jnp.float32)

    def zrow(i, carry):
        for f in range(F // L):
            gb4[0, i, pl.ds(f * L, L)] = zero
        return carry

    lax.fori_loop(0, CHUNK, zrow, 0)
    r0 = s * ROWS_PER_TILE
    for off in range(0, ROWS_PER_TILE, CHUNK):
        sz = min(CHUNK, ROWS_PER_TILE - off)
        pltpu.sync_copy(gb4.at[0, pl.ds(0, sz)], acc.at[pl.ds(r0 + off, sz)])

    @pl.when(s == NS - 1)
    def _zero_tail():
        pltpu.sync_copy(gb4.at[0, pl.ds(0, TAIL_ROWS)],
                        acc.at[pl.ds(NS * ROWS_PER_TILE, TAIL_ROWS)])

    plsc.subcore_barrier()

    for j in range(3):
        idx_load(j, j)
        idx_wait(j, j)
        gather_copy(j).start()
    idx_load(3, 3)

    n_quads = (N_CHUNKS - 4) // 4

    def quad(p, carry):
        i0 = 4 * p
        step(i0, 0, True, True)
        step(i0 + 1, 1, True, True)
        step(i0 + 2, 2, True, True)
        step(i0 + 3, 3, True, True)
        return carry

    lax.fori_loop(0, n_quads, quad, 0)

    for j in range(4 * n_quads, N_CHUNKS):
        step(j, j % NSLOT, j + 3 < N_CHUNKS, j + 4 < N_CHUNKS)

    plsc.subcore_barrier()

    for off in range(0, ROWS_PER_TILE, CHUNK):
        sz = min(CHUNK, ROWS_PER_TILE - off)
        pltpu.sync_copy(acc.at[pl.ds(r0 + off, sz)], gb4.at[0, pl.ds(0, sz)])
        pltpu.sync_copy(gb4.at[0, pl.ds(0, sz)], out.at[c, pl.ds(r0 + off, sz)])

    @pl.when(s == NS - 1)
    def _write_tail():
        t0 = NS * ROWS_PER_TILE
        pltpu.sync_copy(acc.at[pl.ds(t0, TAIL_ROWS)], gb4.at[0, pl.ds(0, TAIL_ROWS)])
        pltpu.sync_copy(gb4.at[0, pl.ds(0, TAIL_ROWS)], out.at[c, pl.ds(t0, TAIL_ROWS)])


def _comb_body(p_ref, b_ref, o_ref):
    o_ref[...] = p_ref[0] + p_ref[1] + b_ref[...]


def _combine(partials, b2):
    return pl.pallas_call(
        _comb_body,
        grid=(10,),
        in_specs=[
            pl.BlockSpec((NC, 1000, F), lambda i: (0, i, 0)),
            pl.BlockSpec((1, F), lambda i: (0, 0)),
        ],
        out_specs=pl.BlockSpec((1000, F), lambda i: (i, 0)),
        out_shape=jax.ShapeDtypeStruct((N_NODES, F), jnp.float32),
    )(partials, b2)


def kernel(input, edge_index, edge_weight, W, b):
    ei = edge_index.astype(jnp.int32)
    row = ei[0]
    col = ei[1]
    sup = _matmul(input, W)
    partials = _spmm(sup, col, row, edge_weight)
    return _combine(partials, b.reshape(1, F))
